# baseline (speedup 1.0000x reference)
import functools

import jax
import jax.numpy as jnp
from jax import lax
from jax.experimental import pallas as pl
from jax.experimental.pallas import tpu as pltpu
from jax.experimental.pallas import tpu_sc as plsc

KS = 5
KKK = KS ** 3
NNODE = 6890
NPAD = 6912
NW = 32
CHUNK = 32
RPT = NPAD // 16
NCLS = 6890

f32 = jnp.float32
i32 = jnp.int32


@functools.lru_cache(maxsize=None)
def _make_edge_pass(Co, TW, with_deg, e_pad):
    epw = e_pad // NW
    nch = epw // CHUNK
    assert nch % 2 == 0
    ngrp = epw // 16
    rpc = 8 * CHUNK

    mesh = plsc.VectorSubcoreMesh(core_axis_name="c", subcore_axis_name="s")

    if with_deg:
        out_type = (jax.ShapeDtypeStruct((2, NPAD, Co), f32),
                    jax.ShapeDtypeStruct((2, NPAD, 16), f32))
    else:
        out_type = jax.ShapeDtypeStruct((2, NPAD, Co), f32)

    scratch = [pltpu.VMEM_SHARED((NPAD, Co), f32)]
    if with_deg:
        scratch.append(pltpu.VMEM_SHARED((NPAD, 16), f32))
    scratch += [
        pltpu.VMEM((rpc, TW), f32),
        pltpu.VMEM((rpc, TW), f32),
        pltpu.VMEM((CHUNK, Co), f32),
        pltpu.VMEM((CHUNK, Co), f32),
        pltpu.VMEM((CHUNK,), i32),
        pltpu.VMEM((CHUNK,), i32),
        pltpu.VMEM((8 * epw,), i32),
        pltpu.VMEM((8 * epw,), f32),
        pltpu.VMEM((epw,), i32),
        pltpu.VMEM((epw,), i32),
        pltpu.VMEM((3 * epw,), f32),
    ]
    if with_deg:
        scratch.append(pltpu.VMEM((CHUNK, 16), f32))
    scratch += [pltpu.SemaphoreType.DMA, pltpu.SemaphoreType.DMA,
                pltpu.SemaphoreType.DMA, pltpu.SemaphoreType.DMA]

    def body(*refs):
        it = iter(refs)
        z_hbm = next(it)
        p_hbm = next(it)
        src_hbm = next(it)
        dst_hbm = next(it)
        zac_hbm = next(it)
        if with_deg:
            zdg_hbm = next(it)
            one_hbm = next(it)
        out_hbm = next(it)
        if with_deg:
            deg_hbm = next(it)
        acc = next(it)
        if with_deg:
            accd = next(it)
        rowsA = next(it)
        rowsB = next(it)
        msgA = next(it)
        msgB = next(it)
        dstA = next(it)
        dstB = next(it)
        gid = next(it)
        bw = next(it)
        srcw = next(it)
        dstw = next(it)
        pvw = next(it)
        if with_deg:
            onesb = next(it)
        semA = next(it)
        semB = next(it)
        semSA = next(it)
        semSB = next(it)

        c = lax.axis_index("c")
        sid = lax.axis_index("s")
        wid = sid * 2 + c
        r0 = sid * RPT
        w0 = wid * epw

        pltpu.sync_copy(zac_hbm.at[pl.ds(r0, RPT), :], acc.at[pl.ds(r0, RPT), :])
        if with_deg:
            pltpu.sync_copy(zdg_hbm.at[pl.ds(r0, RPT), :],
                            accd.at[pl.ds(r0, RPT), :])
            pltpu.sync_copy(one_hbm, onesb)
        pltpu.sync_copy(src_hbm.at[pl.ds(w0, epw)], srcw)
        pltpu.sync_copy(dst_hbm.at[pl.ds(w0, epw)], dstw)
        for d in range(3):
            pltpu.sync_copy(p_hbm.at[pl.ds(d * e_pad + w0, epw)],
                            pvw.at[pl.ds(d * epw, epw)])
        plsc.subcore_barrier()

        gpc = CHUNK // 16

        def basis_body(g, carry):
            t = g // gpc
            gg = g % gpc
            sg = srcw[pl.ds(g * 16, 16)]
            fr = []
            bo = []
            for d in range(3):
                v = pvw[pl.ds(d * epw + g * 16, 16)]
                bi = v.astype(i32)
                fr.append(v - bi.astype(f32))
                bo.append(bi)
            pos0 = t * rpc + gg * 16
            for s in range(8):
                b = None
                idx = None
                stride = 1
                for d in range(3):
                    o = (s >> d) & 1
                    f = fr[d] if o else (1.0 - fr[d])
                    b = f if b is None else b * f
                    kd = jnp.minimum(bo[d] + o, KS - 1)
                    term = kd * stride
                    idx = term if idx is None else idx + term
                    stride *= KS
                bw[pl.ds(pos0 + s * CHUNK, 16)] = b
                gid[pl.ds(pos0 + s * CHUNK, 16)] = sg * KKK + idx
            return carry

        lax.fori_loop(0, ngrp, basis_body, 0)

        def fire(t, rowsX, semX):
            for s in range(8):
                pltpu.async_copy(
                    z_hbm.at[gid.at[pl.ds(t * rpc + s * CHUNK, CHUNK)]],
                    rowsX.at[pl.ds(s * CHUNK, CHUNK), :], semX)

        def drain(t, rowsX, semX):
            for s in range(8):
                pltpu.make_async_copy(
                    z_hbm.at[gid.at[pl.ds(t * rpc + s * CHUNK, CHUNK)]],
                    rowsX.at[pl.ds(s * CHUNK, CHUNK), :], semX).wait()

        def reduce_chunk(t, rowsX, msgX, dstX):
            for g in range(gpc):
                dstX[pl.ds(g * 16, 16)] = dstw[pl.ds(t * CHUNK + g * 16, 16)]

            def group_body(g, carry):
                pos0 = t * rpc + g * 16
                bvecs = [bw[pl.ds(pos0 + s * CHUNK, 16)] for s in range(8)]
                rbase = g * 16
                for eg in range(16):
                    for ccc in range(Co // 16):
                        accv = None
                        for s in range(8):
                            rv = rowsX[s * CHUNK + rbase + eg,
                                       pl.ds(ccc * 16, 16)]
                            term = rv * bvecs[s][eg]
                            accv = term if accv is None else accv + term
                        msgX[rbase + eg, pl.ds(ccc * 16, 16)] = accv
                return carry

            lax.fori_loop(0, gpc, group_body, 0)

        def fire_scatter(msgX, dstX, semSX):
            pltpu.async_copy(msgX, acc.at[dstX], semSX, add=True)
            if with_deg:
                pltpu.async_copy(onesb, accd.at[dstX], semSX, add=True)

        def wait_scatter(msgX, dstX, semSX):
            pltpu.make_async_copy(msgX, acc.at[dstX], semSX).wait()
            if with_deg:
                pltpu.make_async_copy(onesb, accd.at[dstX], semSX).wait()

        fire(0, rowsA, semA)
        fire(1, rowsB, semB)

        def pipe_body(t2, carry):
            tA = t2 * 2
            tB = tA + 1
            drain(tA, rowsA, semA)

            @pl.when(t2 > 0)
            def _():
                wait_scatter(msgA, dstA, semSA)

            reduce_chunk(tA, rowsA, msgA, dstA)

            @pl.when(tA + 2 < nch)
            def _():
                fire(tA + 2, rowsA, semA)

            fire_scatter(msgA, dstA, semSA)

            drain(tB, rowsB, semB)

            @pl.when(t2 > 0)
            def _():
                wait_scatter(msgB, dstB, semSB)

            reduce_chunk(tB, rowsB, msgB, dstB)

            @pl.when(tB + 2 < nch)
            def _():
                fire(tB + 2, rowsB, semB)

            fire_scatter(msgB, dstB, semSB)
            return carry

        lax.fori_loop(0, nch // 2, pipe_body, 0)

        wait_scatter(msgA, dstA, semSA)
        wait_scatter(msgB, dstB, semSB)
        plsc.subcore_barrier()
        pltpu.sync_copy(acc.at[pl.ds(r0, RPT), :],
                        out_hbm.at[c, pl.ds(r0, RPT), :])
        if with_deg:
            pltpu.sync_copy(accd.at[pl.ds(r0, RPT), :],
                            deg_hbm.at[c, pl.ds(r0, RPT), :])

    return pl.kernel(body, out_type=out_type, mesh=mesh,
                     scratch_types=scratch,
                     compiler_params=pltpu.CompilerParams(
                         use_tc_tiling_on_sc=False))


@functools.lru_cache(maxsize=None)
def _make_mm(Cin, KCo):
    def kfn(h_ref, w_ref, o_ref):
        if Cin == 1:
            o_ref[...] = h_ref[...] * w_ref[...]
        else:
            o_ref[...] = jnp.dot(h_ref[...], w_ref[...],
                                 preferred_element_type=f32)

    return pl.pallas_call(
        kfn, grid=(NPAD // 128,),
        in_specs=[pl.BlockSpec((128, Cin), lambda i: (i, 0)),
                  pl.BlockSpec((Cin, KCo), lambda i: (0, 0))],
        out_specs=pl.BlockSpec((128, KCo), lambda i: (i, 0)),
        out_shape=jax.ShapeDtypeStruct((NPAD, KCo), f32))


@functools.lru_cache(maxsize=None)
def _make_epi(Cin, Co):
    def kfn(p_ref, d_ref, h_ref, r_ref, b_ref, o_ref):
        psum = p_ref[0] + p_ref[1]
        deg = d_ref[0, :, 0:1] + d_ref[1, :, 0:1]
        if Cin == 1:
            xr = h_ref[...] * r_ref[...]
        else:
            xr = jnp.dot(h_ref[...], r_ref[...], preferred_element_type=f32)
        val = psum / jnp.maximum(deg, 1.0) + xr + b_ref[...]
        o_ref[...] = jnp.where(val > 0, val,
                               jnp.exp(jnp.minimum(val, 0.0)) - 1.0)

    return pl.pallas_call(
        kfn, grid=(NPAD // 128,),
        in_specs=[pl.BlockSpec((2, 128, Co), lambda i: (0, i, 0)),
                  pl.BlockSpec((2, 128, 16), lambda i: (0, i, 0)),
                  pl.BlockSpec((128, Cin), lambda i: (i, 0)),
                  pl.BlockSpec((Cin, Co), lambda i: (0, 0)),
                  pl.BlockSpec((1, Co), lambda i: (0, 0))],
        out_specs=pl.BlockSpec((128, Co), lambda i: (i, 0)),
        out_shape=jax.ShapeDtypeStruct((NPAD, Co), f32))


def _head_kernel(h_ref, w1_ref, b1_ref, w2_ref, b2_ref, o_ref):
    a = jnp.dot(h_ref[...], w1_ref[...], preferred_element_type=f32) + b1_ref[...]
    a = jnp.where(a > 0, a, jnp.exp(jnp.minimum(a, 0.0)) - 1.0)
    logits = jnp.dot(a, w2_ref[...], preferred_element_type=f32) + b2_ref[...]
    m = jnp.max(logits, axis=1, keepdims=True)
    lse = jnp.log(jnp.sum(jnp.exp(logits - m), axis=1, keepdims=True)) + m
    o_ref[...] = logits - lse


_head = pl.pallas_call(
    _head_kernel, grid=(NPAD // 128,),
    in_specs=[pl.BlockSpec((128, 64), lambda i: (i, 0)),
              pl.BlockSpec((64, 256), lambda i: (0, 0)),
              pl.BlockSpec((1, 256), lambda i: (0, 0)),
              pl.BlockSpec((256, NCLS), lambda i: (0, 0)),
              pl.BlockSpec((1, NCLS), lambda i: (0, 0))],
    out_specs=pl.BlockSpec((128, NCLS), lambda i: (i, 0)),
    out_shape=jax.ShapeDtypeStruct((NNODE, NCLS), f32))


def kernel(x, edge_index, pseudo, W1, r1, b1, W2, r2, b2, W3, r3, b3,
           W4, r4, b4, W5, r5, b5, W6, r6, b6, lw1, lb1, lw2, lb2):
    E = pseudo.shape[0]
    qe = NW * CHUNK * 2
    e_pad = -(-E // qe) * qe
    pad_e = e_pad - E

    src = edge_index[0].astype(i32)
    dst = edge_index[1].astype(i32)
    srcp = jnp.pad(src, (0, pad_e))
    dstp = jnp.pad(dst, (0, pad_e), constant_values=NNODE)
    pq = (pseudo.astype(f32) * (KS - 1)).T
    pflat = jnp.pad(pq, ((0, 0), (0, pad_e))).reshape(-1)
    h = jnp.pad(x.astype(f32), ((0, NPAD - NNODE), (0, 0)))

    zdeg = jnp.zeros((NPAD, 16), f32)
    ones16 = jnp.ones((CHUNK, 16), f32)

    deg2 = None
    for li, (W, r, b) in enumerate([(W1, r1, b1), (W2, r2, b2), (W3, r3, b3),
                                    (W4, r4, b4), (W5, r5, b5), (W6, r6, b6)]):
        Cin, Co = W.shape[1], W.shape[2]
        TW = Co
        Wt = jnp.transpose(W, (1, 0, 2))
        if TW != Co:
            Wt = jnp.pad(Wt, ((0, 0), (0, 0), (0, TW - Co)))
        W2d = Wt.reshape(Cin, KKK * TW)
        z2 = _make_mm(Cin, KKK * TW)(h, W2d).reshape(NPAD * KKK, TW)
        zacc = jnp.zeros((NPAD, Co), f32)
        ep = _make_edge_pass(Co, TW, li == 0, e_pad)
        if li == 0:
            parts, deg2 = ep(z2, pflat, srcp, dstp, zacc, zdeg, ones16)
        else:
            parts = ep(z2, pflat, srcp, dstp, zacc)
        h = _make_epi(Cin, Co)(parts, deg2, h, r, jnp.reshape(b, (1, Co)))

    return _head(h, lw1, jnp.reshape(lb1, (1, 256)), lw2,
                 jnp.reshape(lb2, (1, NCLS)))


# speedup vs baseline: 1.0473x; 1.0473x over previous
import functools

import jax
import jax.numpy as jnp
from jax import lax
from jax.experimental import pallas as pl
from jax.experimental.pallas import tpu as pltpu
from jax.experimental.pallas import tpu_sc as plsc

KS = 5
KKK = KS ** 3
NNODE = 6890
NPAD = 6912
NW = 32
CHUNK = 32
RPT = NPAD // 16
NCLS = 6890

f32 = jnp.float32
i32 = jnp.int32


@functools.lru_cache(maxsize=None)
def _make_edge_pass(Co, TW, with_deg, e_pad):
    epw = e_pad // NW
    nch = epw // CHUNK
    assert nch % 2 == 0
    ngrp = epw // 16
    rpc = 8 * CHUNK

    mesh = plsc.VectorSubcoreMesh(core_axis_name="c", subcore_axis_name="s")

    if with_deg:
        out_type = (jax.ShapeDtypeStruct((2, NPAD, Co), f32),
                    jax.ShapeDtypeStruct((2, NPAD, 16), f32))
    else:
        out_type = jax.ShapeDtypeStruct((2, NPAD, Co), f32)

    scratch = [pltpu.VMEM_SHARED((NPAD, Co), f32)]
    if with_deg:
        scratch.append(pltpu.VMEM_SHARED((NPAD, 16), f32))
    scratch += [
        pltpu.VMEM((rpc, TW), f32),
        pltpu.VMEM((rpc, TW), f32),
        pltpu.VMEM((CHUNK, Co), f32),
        pltpu.VMEM((CHUNK, Co), f32),
        pltpu.VMEM((CHUNK,), i32),
        pltpu.VMEM((CHUNK,), i32),
        pltpu.VMEM((8 * epw,), i32),
        pltpu.VMEM((8 * epw,), f32),
        pltpu.VMEM((epw,), i32),
        pltpu.VMEM((epw,), i32),
        pltpu.VMEM((3 * epw,), f32),
    ]
    if with_deg:
        scratch.append(pltpu.VMEM((CHUNK, 16), f32))
    scratch += [pltpu.SemaphoreType.DMA, pltpu.SemaphoreType.DMA,
                pltpu.SemaphoreType.DMA, pltpu.SemaphoreType.DMA]

    def body(*refs):
        it = iter(refs)
        z_hbm = next(it)
        p_hbm = next(it)
        src_hbm = next(it)
        dst_hbm = next(it)
        zac_hbm = next(it)
        if with_deg:
            zdg_hbm = next(it)
            one_hbm = next(it)
        out_hbm = next(it)
        if with_deg:
            deg_hbm = next(it)
        acc = next(it)
        if with_deg:
            accd = next(it)
        rowsA = next(it)
        rowsB = next(it)
        msgA = next(it)
        msgB = next(it)
        dstA = next(it)
        dstB = next(it)
        gid = next(it)
        bw = next(it)
        srcw = next(it)
        dstw = next(it)
        pvw = next(it)
        if with_deg:
            onesb = next(it)
        semA = next(it)
        semB = next(it)
        semSA = next(it)
        semSB = next(it)

        c = lax.axis_index("c")
        sid = lax.axis_index("s")
        wid = sid * 2 + c
        r0 = sid * RPT
        w0 = wid * epw

        pltpu.sync_copy(zac_hbm.at[pl.ds(r0, RPT), :], acc.at[pl.ds(r0, RPT), :])
        if with_deg:
            pltpu.sync_copy(zdg_hbm.at[pl.ds(r0, RPT), :],
                            accd.at[pl.ds(r0, RPT), :])
            pltpu.sync_copy(one_hbm, onesb)
        pltpu.sync_copy(src_hbm.at[pl.ds(w0, epw)], srcw)
        pltpu.sync_copy(dst_hbm.at[pl.ds(w0, epw)], dstw)
        for d in range(3):
            pltpu.sync_copy(p_hbm.at[pl.ds(d * e_pad + w0, epw)],
                            pvw.at[pl.ds(d * epw, epw)])
        plsc.subcore_barrier()

        gpc = CHUNK // 16

        def basis_body(g, carry):
            t = g // gpc
            gg = g % gpc
            sg = srcw[pl.ds(g * 16, 16)]
            fr = []
            bo = []
            for d in range(3):
                v = pvw[pl.ds(d * epw + g * 16, 16)]
                bi = v.astype(i32)
                fr.append(v - bi.astype(f32))
                bo.append(bi)
            pos0 = t * rpc + gg * 16
            for s in range(8):
                b = None
                idx = None
                stride = 1
                for d in range(3):
                    o = (s >> d) & 1
                    f = fr[d] if o else (1.0 - fr[d])
                    b = f if b is None else b * f
                    kd = jnp.minimum(bo[d] + o, KS - 1)
                    term = kd * stride
                    idx = term if idx is None else idx + term
                    stride *= KS
                bw[pl.ds(pos0 + s * CHUNK, 16)] = b
                gid[pl.ds(pos0 + s * CHUNK, 16)] = sg * KKK + idx
            return carry

        lax.fori_loop(0, ngrp, basis_body, 0)

        def fire(t, rowsX, semX):
            for s in range(8):
                pltpu.async_copy(
                    z_hbm.at[gid.at[pl.ds(t * rpc + s * CHUNK, CHUNK)]],
                    rowsX.at[pl.ds(s * CHUNK, CHUNK), :], semX)

        def drain(t, rowsX, semX):
            for s in range(8):
                pltpu.make_async_copy(
                    z_hbm.at[gid.at[pl.ds(t * rpc + s * CHUNK, CHUNK)]],
                    rowsX.at[pl.ds(s * CHUNK, CHUNK), :], semX).wait()

        def reduce_chunk(t, rowsX, msgX, dstX):
            for g in range(gpc):
                dstX[pl.ds(g * 16, 16)] = dstw[pl.ds(t * CHUNK + g * 16, 16)]

            def group_body(g, carry):
                pos0 = t * rpc + g * 16
                bvecs = [bw[pl.ds(pos0 + s * CHUNK, 16)] for s in range(8)]
                rbase = g * 16
                for eg in range(16):
                    for ccc in range(Co // 16):
                        accv = None
                        for s in range(8):
                            rv = rowsX[s * CHUNK + rbase + eg,
                                       pl.ds(ccc * 16, 16)]
                            term = rv * bvecs[s][eg]
                            accv = term if accv is None else accv + term
                        msgX[rbase + eg, pl.ds(ccc * 16, 16)] = accv
                return carry

            lax.fori_loop(0, gpc, group_body, 0)

        def fire_scatter(msgX, dstX, semSX):
            pltpu.async_copy(msgX, acc.at[dstX], semSX, add=True)
            if with_deg:
                pltpu.async_copy(onesb, accd.at[dstX], semSX, add=True)

        def wait_scatter(msgX, dstX, semSX):
            pltpu.make_async_copy(msgX, acc.at[dstX], semSX).wait()
            if with_deg:
                pltpu.make_async_copy(onesb, accd.at[dstX], semSX).wait()

        fire(0, rowsA, semA)
        fire(1, rowsB, semB)

        def pipe_body(t2, carry):
            tA = t2 * 2
            tB = tA + 1
            drain(tA, rowsA, semA)

            @pl.when(t2 > 0)
            def _():
                wait_scatter(msgA, dstA, semSA)

            reduce_chunk(tA, rowsA, msgA, dstA)

            @pl.when(tA + 2 < nch)
            def _():
                fire(tA + 2, rowsA, semA)

            fire_scatter(msgA, dstA, semSA)

            drain(tB, rowsB, semB)

            @pl.when(t2 > 0)
            def _():
                wait_scatter(msgB, dstB, semSB)

            reduce_chunk(tB, rowsB, msgB, dstB)

            @pl.when(tB + 2 < nch)
            def _():
                fire(tB + 2, rowsB, semB)

            fire_scatter(msgB, dstB, semSB)
            return carry

        lax.fori_loop(0, nch // 2, pipe_body, 0)

        wait_scatter(msgA, dstA, semSA)
        wait_scatter(msgB, dstB, semSB)
        plsc.subcore_barrier()
        pltpu.sync_copy(acc.at[pl.ds(r0, RPT), :],
                        out_hbm.at[c, pl.ds(r0, RPT), :])
        if with_deg:
            pltpu.sync_copy(accd.at[pl.ds(r0, RPT), :],
                            deg_hbm.at[c, pl.ds(r0, RPT), :])

    return pl.kernel(body, out_type=out_type, mesh=mesh,
                     scratch_types=scratch,
                     compiler_params=pltpu.CompilerParams(
                         use_tc_tiling_on_sc=False))


@functools.lru_cache(maxsize=None)
def _make_mm(Cin, KCo):
    def kfn(h_ref, w_ref, o_ref):
        if Cin == 1:
            o_ref[...] = h_ref[...] * w_ref[...]
        else:
            o_ref[...] = jnp.dot(h_ref[...], w_ref[...],
                                 preferred_element_type=f32)

    return pl.pallas_call(
        kfn, grid=(NPAD // 128,),
        in_specs=[pl.BlockSpec((128, Cin), lambda i: (i, 0)),
                  pl.BlockSpec((Cin, KCo), lambda i: (0, 0))],
        out_specs=pl.BlockSpec((128, KCo), lambda i: (i, 0)),
        out_shape=jax.ShapeDtypeStruct((NPAD, KCo), f32))


def _elu(v):
    return jnp.where(v > 0, v, jnp.exp(jnp.minimum(v, 0.0)) - 1.0)


def _epi_val(p_ref, d_ref, h_ref, r_ref, b_ref, Cin):
    psum = p_ref[0] + p_ref[1]
    deg = d_ref[0, :, 0:1] + d_ref[1, :, 0:1]
    if Cin == 1:
        xr = h_ref[...] * r_ref[...]
    else:
        xr = jnp.dot(h_ref[...], r_ref[...], preferred_element_type=f32)
    return _elu(psum / jnp.maximum(deg, 1.0) + xr + b_ref[...])


@functools.lru_cache(maxsize=None)
def _make_epimm(Cin, Co, KCo):
    def kfn(p_ref, d_ref, h_ref, r_ref, b_ref, w_ref, ho_ref, z_ref):
        hblk = _epi_val(p_ref, d_ref, h_ref, r_ref, b_ref, Cin)
        ho_ref[...] = hblk
        z_ref[...] = jnp.dot(hblk, w_ref[...], preferred_element_type=f32)

    return pl.pallas_call(
        kfn, grid=(NPAD // 128,),
        in_specs=[pl.BlockSpec((2, 128, Co), lambda i: (0, i, 0)),
                  pl.BlockSpec((2, 128, 16), lambda i: (0, i, 0)),
                  pl.BlockSpec((128, Cin), lambda i: (i, 0)),
                  pl.BlockSpec((Cin, Co), lambda i: (0, 0)),
                  pl.BlockSpec((1, Co), lambda i: (0, 0)),
                  pl.BlockSpec((Co, KCo), lambda i: (0, 0))],
        out_specs=[pl.BlockSpec((128, Co), lambda i: (i, 0)),
                   pl.BlockSpec((128, KCo), lambda i: (i, 0))],
        out_shape=[jax.ShapeDtypeStruct((NPAD, Co), f32),
                   jax.ShapeDtypeStruct((NPAD, KCo), f32)])


def _head_kernel(p_ref, d_ref, h_ref, r_ref, b_ref,
                 w1_ref, b1_ref, w2_ref, b2_ref, o_ref):
    h6 = _epi_val(p_ref, d_ref, h_ref, r_ref, b_ref, 64)
    a = _elu(jnp.dot(h6, w1_ref[...], preferred_element_type=f32) + b1_ref[...])
    logits = jnp.dot(a, w2_ref[...], preferred_element_type=f32) + b2_ref[...]
    m = jnp.max(logits, axis=1, keepdims=True)
    lse = jnp.log(jnp.sum(jnp.exp(logits - m), axis=1, keepdims=True)) + m
    o_ref[...] = logits - lse


_head = pl.pallas_call(
    _head_kernel, grid=(NPAD // 128,),
    in_specs=[pl.BlockSpec((2, 128, 64), lambda i: (0, i, 0)),
              pl.BlockSpec((2, 128, 16), lambda i: (0, i, 0)),
              pl.BlockSpec((128, 64), lambda i: (i, 0)),
              pl.BlockSpec((64, 64), lambda i: (0, 0)),
              pl.BlockSpec((1, 64), lambda i: (0, 0)),
              pl.BlockSpec((64, 256), lambda i: (0, 0)),
              pl.BlockSpec((1, 256), lambda i: (0, 0)),
              pl.BlockSpec((256, NCLS), lambda i: (0, 0)),
              pl.BlockSpec((1, NCLS), lambda i: (0, 0))],
    out_specs=pl.BlockSpec((128, NCLS), lambda i: (i, 0)),
    out_shape=jax.ShapeDtypeStruct((NNODE, NCLS), f32))


def kernel(x, edge_index, pseudo, W1, r1, b1, W2, r2, b2, W3, r3, b3,
           W4, r4, b4, W5, r5, b5, W6, r6, b6, lw1, lb1, lw2, lb2):
    E = pseudo.shape[0]
    qe = NW * CHUNK * 2
    e_pad = -(-E // qe) * qe
    pad_e = e_pad - E

    src = edge_index[0].astype(i32)
    dst = edge_index[1].astype(i32)
    srcp = jnp.pad(src, (0, pad_e))
    dstp = jnp.pad(dst, (0, pad_e), constant_values=NNODE)
    pq = (pseudo.astype(f32) * (KS - 1)).T
    pflat = jnp.pad(pq, ((0, 0), (0, pad_e))).reshape(-1)
    h = jnp.pad(x.astype(f32), ((0, NPAD - NNODE), (0, 0)))

    zdeg = jnp.zeros((NPAD, 16), f32)
    ones16 = jnp.ones((CHUNK, 16), f32)

    Ws = [(W1, r1, b1), (W2, r2, b2), (W3, r3, b3),
          (W4, r4, b4), (W5, r5, b5), (W6, r6, b6)]

    def w2d(W):
        Cin, Co = W.shape[1], W.shape[2]
        return jnp.transpose(W, (1, 0, 2)).reshape(Cin, KKK * Co)

    def run_sc(z2, Co, first):
        zacc = jnp.zeros((NPAD, Co), f32)
        ep = _make_edge_pass(Co, Co, first, e_pad)
        if first:
            return ep(z2, pflat, srcp, dstp, zacc, zdeg, ones16)
        return ep(z2, pflat, srcp, dstp, zacc)

    z2 = _make_mm(1, KKK * 32)(h, w2d(W1)).reshape(NPAD * KKK, 32)
    parts, deg2 = run_sc(z2, 32, True)
    hprev = h
    for li in range(1, 6):
        _, rp, bp = Ws[li - 1]
        Cin_p, Co_p = Ws[li - 1][0].shape[1], Ws[li - 1][0].shape[2]
        Wn = Ws[li][0]
        Co_n = Wn.shape[2]
        hprev, z = _make_epimm(Cin_p, Co_p, KKK * Co_n)(
            parts, deg2, hprev, rp, jnp.reshape(bp, (1, Co_p)), w2d(Wn))
        parts = run_sc(z.reshape(NPAD * KKK, Co_n), Co_n, False)

    return _head(parts, deg2, hprev, r6, jnp.reshape(b6, (1, 64)),
                 lw1, jnp.reshape(lb1, (1, 256)), lw2,
                 jnp.reshape(lb2, (1, NCLS)))
